# baseline (speedup 1.0000x reference)
import functools

import jax
import jax.numpy as jnp
from jax import lax
from jax.experimental import pallas as pl
from jax.experimental.pallas import tpu as pltpu
from jax.experimental.pallas import tpu_sc as plsc

_N_FRAGS = 16384
_IN_DIM = 4096
_N_EXPERTS = 64
_TOP_K = 8
_BLOCK_COLS = 1024
_GRID = _N_FRAGS // _BLOCK_COLS
_LANES = 16
_NW = 32
_RPW = _N_FRAGS // _NW
_NGROUPS = _RPW // _LANES


def _logits_block(x_ref, w_ref, out_ref):
    out_ref[...] = jax.lax.dot_general(
        w_ref[...].astype(jnp.bfloat16),
        x_ref[...].astype(jnp.bfloat16),
        dimension_numbers=(((1,), (1,)), ((), ())),
        preferred_element_type=jnp.float32,
    )


def _tc_logits_t(x, W):
    return pl.pallas_call(
        _logits_block,
        grid=(_GRID,),
        in_specs=[
            pl.BlockSpec((_BLOCK_COLS, _IN_DIM), lambda i: (i, 0)),
            pl.BlockSpec((_N_EXPERTS, _IN_DIM), lambda i: (0, 0)),
        ],
        out_specs=pl.BlockSpec((_N_EXPERTS, _BLOCK_COLS), lambda i: (0, i)),
        out_shape=jax.ShapeDtypeStruct((_N_EXPERTS, _N_FRAGS), jnp.float32),
        compiler_params=pltpu.CompilerParams(
            dimension_semantics=("parallel",),
        ),
    )(x, W)


_MESH = plsc.VectorSubcoreMesh(core_axis_name="c", subcore_axis_name="s")


@functools.partial(
    pl.kernel,
    mesh=_MESH,
    out_type=[
        jax.ShapeDtypeStruct((_N_EXPERTS, _N_FRAGS), jnp.float32),
        jax.ShapeDtypeStruct((_NW, _N_EXPERTS, _LANES), jnp.float32),
    ],
    scratch_types=[
        pltpu.VMEM((_N_EXPERTS, _RPW), jnp.float32),
        pltpu.VMEM((_N_EXPERTS, _RPW), jnp.float32),
        pltpu.VMEM((_N_EXPERTS, _LANES), jnp.float32),
    ],
)
def _sc_route(lt_hbm, wt_hbm, part_hbm, lbuf, wbuf, accbuf):
    wid = lax.axis_index("s") * 2 + lax.axis_index("c")
    base = wid * _RPW
    pltpu.sync_copy(lt_hbm.at[:, pl.ds(base, _RPW)], lbuf)

    neg_inf = jnp.float32(-jnp.inf)
    zeros = jnp.zeros((_LANES,), jnp.float32)

    for e in range(_N_EXPERTS):
        accbuf[e] = zeros

    def group(g, carry):
        col = g * _LANES
        sl = pl.ds(col, _LANES)
        work = [lbuf[e, sl] for e in range(_N_EXPERTS)]
        row_max = None
        thresh = None
        for t in range(_TOP_K):
            m = work[0]
            for e in range(1, _N_EXPERTS):
                m = jnp.maximum(m, work[e])
            if t == 0:
                row_max = m
            thresh = m
            if t < _TOP_K - 1:
                for e in range(_N_EXPERTS):
                    work[e] = jnp.where(work[e] == m, neg_inf, work[e])
        s = zeros
        ev = []
        for e in range(_N_EXPERTS):
            le = lbuf[e, sl]
            x = jnp.where(le >= thresh, jnp.exp(le - row_max), 0.0)
            ev.append(x)
            s = s + x
        inv = 1.0 / s
        for e in range(_N_EXPERTS):
            w = ev[e] * inv
            wbuf[e, sl] = w
            accbuf[e] = accbuf[e] + w
        return carry

    lax.fori_loop(0, _NGROUPS, group, 0)

    pltpu.sync_copy(wbuf, wt_hbm.at[:, pl.ds(base, _RPW)])
    pltpu.sync_copy(accbuf, part_hbm.at[wid])


@functools.partial(jax.jit)
def kernel(x, W):
    lt = _tc_logits_t(x, W)
    wt, parts = _sc_route(lt)
    weights = wt.T
    load = parts.sum(axis=(0, 2)) * (1.0 / _N_FRAGS)
    return weights, load
